# baseline (speedup 1.0000x reference)
import functools

import jax
import jax.numpy as jnp
from jax import lax
from jax.experimental import pallas as pl
from jax.experimental.pallas import tpu as pltpu
from jax.experimental.pallas import tpu_sc as plsc

D_MODEL = 128
HALF = 64
N_NUM = 8

NC = 2
NS = 16
NW = NC * NS

GATHER_CHUNK = 400
N_SEG = 4


def _proj_body(t_ref, wo_ref, p_ref):
    p_ref[...] = jnp.dot(t_ref[...], wo_ref[...][:HALF],
                         preferred_element_type=jnp.float32,
                         precision=lax.Precision.HIGHEST)


def _project_table(table, W_out, blk=2000):
    v = table.shape[0]
    return pl.pallas_call(
        _proj_body,
        grid=(v // blk,),
        in_specs=[
            pl.BlockSpec((blk, HALF), lambda i: (i, 0)),
            pl.BlockSpec((D_MODEL, D_MODEL), lambda i: (0, 0)),
        ],
        out_specs=pl.BlockSpec((blk, D_MODEL), lambda i: (i, 0)),
        out_shape=jax.ShapeDtypeStruct((v, D_MODEL), jnp.float32),
    )(table, W_out)


def _sc_gather_fn(n_tokens):
    b_per_w = n_tokens // NW
    n_chunks = b_per_w // GATHER_CHUNK

    mesh = plsc.VectorSubcoreMesh(core_axis_name="c", subcore_axis_name="s")

    @functools.partial(
        pl.kernel,
        mesh=mesh,
        out_type=jax.ShapeDtypeStruct((n_tokens, D_MODEL), jnp.float32),
        scratch_types=[
            pltpu.VMEM((GATHER_CHUNK,), jnp.int32),
            pltpu.VMEM((GATHER_CHUNK, D_MODEL), jnp.float32),
            pltpu.SemaphoreType.DMA,
        ],
    )
    def gather_k(table_hbm, idx_hbm, out_hbm, idx_v, rows_v, sem):
        wid = lax.axis_index("s") * NC + lax.axis_index("c")
        base = wid * b_per_w

        def body(i, carry):
            off = pl.multiple_of(base + i * GATHER_CHUNK, GATHER_CHUNK)
            pltpu.sync_copy(idx_hbm.at[pl.ds(off, GATHER_CHUNK)], idx_v)
            pltpu.async_copy(table_hbm.at[idx_v], rows_v, sem).wait()
            pltpu.sync_copy(rows_v, out_hbm.at[pl.ds(off, GATHER_CHUNK)])
            return carry

        lax.fori_loop(0, n_chunks, body, 0, unroll=False)

    return gather_k


def _tail_compute(g, nf, wn, bn, wo, bo, gm, bt):
    wo_b = wo[HALF:]
    wc = jnp.dot(wn, wo_b, preferred_element_type=jnp.float32,
                 precision=lax.Precision.HIGHEST)
    bc = jnp.dot(bn, wo_b, preferred_element_type=jnp.float32,
                 precision=lax.Precision.HIGHEST) + bo
    contrib = jnp.dot(nf, wc, preferred_element_type=jnp.float32,
                      precision=lax.Precision.HIGHEST)
    out = g + contrib + bc
    mean = jnp.mean(out, axis=-1, keepdims=True)
    cent = out - mean
    var = jnp.mean(cent * cent, axis=-1, keepdims=True)
    xhat = cent * lax.rsqrt(var + 1e-5)
    return xhat * gm + bt


def _tail_body(g_ref, nf_ref, wn_ref, bn_ref, wo_ref, bo_ref, gm_ref,
               bt_ref, o_ref):
    o_ref[...] = _tail_compute(g_ref[...], nf_ref[...], wn_ref[...],
                               bn_ref[...], wo_ref[...], bo_ref[...],
                               gm_ref[...], bt_ref[...])


def _tail_body_acc(acc_ref, g_ref, nf_ref, wn_ref, bn_ref, wo_ref, bo_ref,
                   gm_ref, bt_ref, o_ref):
    del acc_ref
    o_ref[...] = _tail_compute(g_ref[...], nf_ref[...], wn_ref[...],
                               bn_ref[...], wo_ref[...], bo_ref[...],
                               gm_ref[...], bt_ref[...])


def _tc_tail_seg(acc, gathered_seg, nf_seg, W_num, b_num, W_out, b_out,
                 gamma, beta, n_tokens, seg, tok_blk=4096):
    seg_tokens = gathered_seg.shape[0]
    blk0 = seg * (seg_tokens // tok_blk)
    specs = [
        pl.BlockSpec((tok_blk, D_MODEL), lambda i: (i, 0)),
        pl.BlockSpec((tok_blk, N_NUM), lambda i: (i, 0)),
        pl.BlockSpec((N_NUM, HALF), lambda i: (0, 0)),
        pl.BlockSpec((1, HALF), lambda i: (0, 0)),
        pl.BlockSpec((D_MODEL, D_MODEL), lambda i: (0, 0)),
        pl.BlockSpec((1, D_MODEL), lambda i: (0, 0)),
        pl.BlockSpec((1, D_MODEL), lambda i: (0, 0)),
        pl.BlockSpec((1, D_MODEL), lambda i: (0, 0)),
    ]
    args = (gathered_seg, nf_seg, W_num, b_num, W_out, b_out, gamma, beta)
    if acc is None:
        body, aliases = _tail_body, {}
    else:
        body, aliases = _tail_body_acc, {0: 0}
        specs = [pl.BlockSpec(memory_space=pl.ANY)] + specs
        args = (acc,) + args
    return pl.pallas_call(
        body,
        grid=(seg_tokens // tok_blk,),
        in_specs=specs,
        out_specs=pl.BlockSpec((tok_blk, D_MODEL), lambda i: (blk0 + i, 0)),
        out_shape=jax.ShapeDtypeStruct((n_tokens, D_MODEL), jnp.float32),
        input_output_aliases=aliases,
    )(*args)


def kernel(event_types, numerical_features, event_table, W_num, b_num,
           W_out, b_out, gamma, beta):
    B, L = event_types.shape
    n_tokens = B * L
    seg_tokens = n_tokens // N_SEG
    idx = event_types.reshape(n_tokens).astype(jnp.int32)
    proj = _project_table(event_table, W_out)
    nf = numerical_features.reshape(n_tokens, N_NUM)
    bn = b_num.reshape(1, HALF)
    bo = b_out.reshape(1, D_MODEL)
    gm = gamma.reshape(1, D_MODEL)
    bt = beta.reshape(1, D_MODEL)
    gather = _sc_gather_fn(seg_tokens)
    acc = None
    for s in range(N_SEG):
        sl = slice(s * seg_tokens, (s + 1) * seg_tokens)
        g_s = gather(proj, idx[sl])
        acc = _tc_tail_seg(acc, g_s, nf[sl], W_num, bn, W_out, bo, gm, bt,
                           n_tokens, s)
    return acc.reshape(B, L, D_MODEL)
